# Initial kernel scaffold; baseline (speedup 1.0000x reference)
import jax
import jax.numpy as jnp
from jax.experimental import pallas as pl


def kernel(q, k):
    raise NotImplementedError("write your pallas kernel here")


# speedup vs baseline: 6.9251x; 6.9251x over previous
import functools

import jax
import jax.numpy as jnp
from jax import lax
from jax.experimental import pallas as pl
from jax.experimental.pallas import tpu as pltpu
from jax.experimental.pallas import tpu_sc as plsc

B = 4
N = 2048
D = 1024
KI = 512
KR = 32
RB = 256
NRB = N // RB
RROWS = 16
NEG = -3.0e38
BIGI = 1 << 30


def _scores_diag_body(q_ref, k_ref, s_ref, st_ref):
    qb = q_ref[0]
    kb = k_ref[0]
    s = lax.dot_general(qb, kb, (((1,), (1,)), ((), ())),
                        preferred_element_type=jnp.float32)
    s_ref[0] = s
    rb = pl.program_id(1)
    rows = lax.broadcasted_iota(jnp.int32, (RB, N), 0) + rb * RB
    cols = lax.broadcasted_iota(jnp.int32, (RB, N), 1)
    m = jnp.max(s, axis=1, keepdims=True)
    d = jnp.max(jnp.where(rows == cols, s, NEG), axis=1, keepdims=True)
    st_ref[0, 0] = jnp.concatenate([m, d], axis=1)


def _topk_diag_body(m_ref, d_ref, z_ref, tk_ref, tkc_ref):
    cols = lax.broadcasted_iota(jnp.int32, (B, N), 1)
    dp = jnp.exp(d_ref[...] - m_ref[...]) / z_ref[...]

    def phase_a(_, carry):
        v, sel = carry
        m = jnp.max(v, axis=1, keepdims=True)
        first = jnp.min(jnp.where(v == m, cols, N), axis=1, keepdims=True)
        hit = cols == first
        return jnp.where(hit, NEG, v), jnp.where(hit, 1, sel)

    sel0 = jnp.zeros((B, N), jnp.int32)
    _, sel = lax.fori_loop(0, KI, phase_a, (dp, sel0))

    work0 = jnp.where(sel == 1, cols, BIGI)
    slots = lax.broadcasted_iota(jnp.int32, (B, KI), 1)
    slots3 = lax.broadcasted_iota(jnp.int32, (B, KI, 1), 1)

    def phase_b(j, carry):
        work, tk, tkc = carry
        cmin = jnp.min(work, axis=1, keepdims=True)
        tk = jnp.where(slots == j, cmin, tk)
        tkc = jnp.where(slots3 == j, cmin[:, :, None], tkc)
        return jnp.where(work == cmin, BIGI, work), tk, tkc

    _, tk, tkc = lax.fori_loop(
        0, KI, phase_b,
        (work0, jnp.zeros((B, KI), jnp.int32),
         jnp.zeros((B, KI, 1), jnp.int32)))
    tk_ref[...] = tk
    tkc_ref[...] = tkc


def _rel_topk_body(qtk_ref, ktk_ref, tk_ref, tkc_ref, st_ref, c_ref, o_ref):
    qt = qtk_ref[0]
    kt = ktk_ref[0]
    s = lax.dot_general(qt, kt, (((1,), (1,)), ((), ())),
                        preferred_element_type=jnp.float32)
    rows = lax.broadcasted_iota(jnp.int32, (KI, KI), 0)
    cols = lax.broadcasted_iota(jnp.int32, (KI, KI), 1)
    colsn = lax.broadcasted_iota(jnp.int32, (KI, N), 1)
    oh = (jnp.broadcast_to(tkc_ref[0], (KI, N)) == colsn).astype(jnp.float32)
    mz = lax.dot_general(oh, st_ref[0], (((1,), (0,)), ((), ())),
                         preferred_element_type=jnp.float32,
                         precision=lax.Precision.HIGHEST)
    p = jnp.exp(s - mz[:, 0:1]) / mz[:, 1:2]
    p = jnp.where(rows == cols, 1.0e9, p)
    tkb = jnp.broadcast_to(tk_ref[0], (KI, KI))

    def phase_a(_, carry):
        v, sel = carry
        m = jnp.max(v, axis=1, keepdims=True)
        first = jnp.min(jnp.where(v == m, cols, KI), axis=1, keepdims=True)
        hit = cols == first
        return jnp.where(hit, NEG, v), jnp.where(hit, 1, sel)

    _, sel = lax.fori_loop(0, KR, phase_a, (p, jnp.zeros((KI, KI), jnp.int32)))

    workc0 = jnp.where(sel == 1, cols, BIGI)
    worko0 = jnp.where(sel == 1, tkb, BIGI)
    slots = lax.broadcasted_iota(jnp.int32, (KI, KR), 1)

    def phase_b(j, carry):
        workc, worko, cmat, omat = carry
        cmin = jnp.min(workc, axis=1, keepdims=True)
        omin = jnp.min(worko, axis=1, keepdims=True)
        cmat = jnp.where(slots == j, cmin, cmat)
        omat = jnp.where(slots == j, omin, omat)
        rm = workc == cmin
        return (jnp.where(rm, BIGI, workc), jnp.where(rm, BIGI, worko),
                cmat, omat)

    z = jnp.zeros((KI, KR), jnp.int32)
    _, _, cmat, omat = lax.fori_loop(0, KR, phase_b,
                                     (workc0, worko0, z, z))
    c_ref[0] = cmat
    o_ref[0] = omat


def _relnorm_body(qtk_ref, c_ref, out_ref):
    rb = pl.program_id(1)
    qt = qtk_ref[0]
    cb = c_ref[0]
    t = lax.broadcasted_iota(jnp.int32, (RROWS, KR, KI), 2)
    onehot = (cb[:, :, None] == t).astype(jnp.float32).reshape(RROWS * KR, KI)
    obj = lax.dot_general(onehot, qt, (((1,), (0,)), ((), ())),
                          preferred_element_type=jnp.float32)
    qs = qtk_ref[0, pl.ds(rb * RROWS, RROWS)]
    subj = jnp.broadcast_to(qs[:, None, :],
                            (RROWS, KR, D)).reshape(RROWS * KR, D)
    rel = subj + obj
    mean = jnp.mean(rel, axis=1, keepdims=True)
    cen = rel - mean
    var = jnp.mean(cen * cen, axis=1, keepdims=True)
    out_ref[0] = cen / jnp.sqrt(var + 1e-5)


def _sc_gather(qf, kf, absidx):
    info = plsc.get_sparse_core_info()
    nc = info.num_cores
    nw = nc * info.num_subcores
    bpw = (B * KI) // nw
    mesh = plsc.VectorSubcoreMesh(core_axis_name="c", subcore_axis_name="s")

    @functools.partial(
        pl.kernel, mesh=mesh,
        out_type=[jax.ShapeDtypeStruct((B * KI, D), jnp.float32),
                  jax.ShapeDtypeStruct((B * KI, D), jnp.float32)],
        scratch_types=[pltpu.VMEM((bpw,), jnp.int32),
                       pltpu.VMEM((bpw, D), jnp.float32),
                       pltpu.SemaphoreType.DMA],
    )
    def gath(q_hbm, k_hbm, idx_hbm, qo_hbm, ko_hbm, idx_v, rows_v, sem):
        wid = lax.axis_index("s") * nc + lax.axis_index("c")
        base = wid * bpw
        pltpu.sync_copy(idx_hbm.at[pl.ds(base, bpw)], idx_v)
        pltpu.async_copy(q_hbm.at[idx_v], rows_v, sem).wait()
        pltpu.sync_copy(rows_v, qo_hbm.at[pl.ds(base, bpw)])
        pltpu.async_copy(k_hbm.at[idx_v], rows_v, sem).wait()
        pltpu.sync_copy(rows_v, ko_hbm.at[pl.ds(base, bpw)])

    return gath(qf, kf, absidx)


def kernel(q, k):
    scores, st = pl.pallas_call(
        _scores_diag_body,
        grid=(B, NRB),
        in_specs=[pl.BlockSpec((1, RB, D), lambda b, r: (b, r, 0)),
                  pl.BlockSpec((1, N, D), lambda b, r: (b, 0, 0))],
        out_specs=[pl.BlockSpec((1, RB, N), lambda b, r: (b, r, 0)),
                   pl.BlockSpec((1, 1, RB, 2), lambda b, r: (b, r, 0, 0))],
        out_shape=[jax.ShapeDtypeStruct((B, N, N), jnp.float32),
                   jax.ShapeDtypeStruct((B, NRB, RB, 2), jnp.float32)],
    )(q, k)
    st = st.reshape(B, N, 2)
    m = st[..., 0]
    d = st[..., 1]
    z = jnp.sum(jnp.exp(scores - m[:, :, None]), axis=-1)

    tk, tkc = pl.pallas_call(
        _topk_diag_body,
        out_shape=[jax.ShapeDtypeStruct((B, KI), jnp.int32),
                   jax.ShapeDtypeStruct((B, KI, 1), jnp.int32)],
    )(m, d, z)

    absidx = (tk + N * jnp.arange(B, dtype=jnp.int32)[:, None]).reshape(B * KI)
    qtk_f, ktk_f = _sc_gather(q.reshape(B * N, D), k.reshape(B * N, D), absidx)
    qtk = qtk_f.reshape(B, KI, D)
    ktk = ktk_f.reshape(B, KI, D)

    c_ids, obj = pl.pallas_call(
        _rel_topk_body,
        grid=(B,),
        in_specs=[pl.BlockSpec((1, KI, D), lambda b: (b, 0, 0)),
                  pl.BlockSpec((1, KI, D), lambda b: (b, 0, 0)),
                  pl.BlockSpec((1, 1, KI), lambda b: (b, 0, 0)),
                  pl.BlockSpec((1, KI, 1), lambda b: (b, 0, 0)),
                  pl.BlockSpec((1, N, 2), lambda b: (b, 0, 0))],
        out_specs=[pl.BlockSpec((1, KI, KR), lambda b: (b, 0, 0)),
                   pl.BlockSpec((1, KI, KR), lambda b: (b, 0, 0))],
        out_shape=[jax.ShapeDtypeStruct((B, KI, KR), jnp.int32),
                   jax.ShapeDtypeStruct((B, KI, KR), jnp.int32)],
    )(qtk, ktk, tk.reshape(B, 1, KI), tkc, jnp.stack([m, z], axis=-1))

    rel_norm = pl.pallas_call(
        _relnorm_body,
        grid=(B, KI // RROWS),
        in_specs=[pl.BlockSpec((1, KI, D), lambda b, r: (b, 0, 0)),
                  pl.BlockSpec((1, RROWS, KR), lambda b, r: (b, r, 0))],
        out_specs=pl.BlockSpec((1, RROWS * KR, D), lambda b, r: (b, r, 0)),
        out_shape=jax.ShapeDtypeStruct((B, KI * KR, D), jnp.float32),
    )(qtk, c_ids)

    subj = jnp.broadcast_to(tk[:, :, None], (B, KI, KR)).reshape(B, KI * KR)
    objf = obj.reshape(B, KI * KR)
    bids = jnp.broadcast_to(jnp.arange(B, dtype=jnp.int32)[:, None],
                            (B, KI * KR))
    soi = jnp.stack([bids, subj, objf], axis=-1)
    return scores, soi, rel_norm


# speedup vs baseline: 10.9129x; 1.5758x over previous
import functools

import jax
import jax.numpy as jnp
from jax import lax
from jax.experimental import pallas as pl
from jax.experimental.pallas import tpu as pltpu
from jax.experimental.pallas import tpu_sc as plsc

B = 4
N = 2048
D = 1024
KI = 512
KR = 32
RB = 256
NRB = N // RB
RROWS = 16
NEG = -3.0e38
BIGI = 1 << 30


def _scores_diag_body(q_ref, k_ref, s_ref, st_ref):
    qb = q_ref[0]
    kb = k_ref[0]
    s = lax.dot_general(qb, kb, (((1,), (1,)), ((), ())),
                        preferred_element_type=jnp.float32)
    s_ref[0] = s
    rb = pl.program_id(1)
    rows = lax.broadcasted_iota(jnp.int32, (RB, N), 0) + rb * RB
    cols = lax.broadcasted_iota(jnp.int32, (RB, N), 1)
    m = jnp.max(s, axis=1, keepdims=True)
    d = jnp.max(jnp.where(rows == cols, s, NEG), axis=1, keepdims=True)
    st_ref[0, 0] = jnp.concatenate([m, d], axis=1)


def _lane_cumsum(x, width):
    cols = lax.broadcasted_iota(jnp.int32, x.shape, x.ndim - 1)
    s = 1
    while s < width:
        x = x + jnp.where(cols >= s, jnp.roll(x, s, axis=-1), 0)
        s *= 2
    return x


def _topk_diag_body(m_ref, d_ref, z_ref, tkc_ref):
    dp = jnp.exp(d_ref[0] - m_ref[0]) / z_ref[0]
    key = lax.bitcast_convert_type(dp, jnp.int32)

    def bit_iter(i, carry):
        prefix, kk = carry
        bit = 30 - i
        cand = prefix | jnp.left_shift(jnp.int32(1), bit)
        c = jnp.sum((lax.shift_right_arithmetic(key, bit)
                     == lax.shift_right_arithmetic(cand, bit)).astype(jnp.int32),
                    axis=1, keepdims=True)
        take = c >= kk
        return jnp.where(take, cand, prefix), jnp.where(take, kk, kk - c)

    prefix0 = jnp.zeros((1, 1), jnp.int32)
    kk0 = jnp.full((1, 1), KI, jnp.int32)
    t, _ = lax.fori_loop(0, 31, bit_iter, (prefix0, kk0))

    gt = key > t
    eq = key == t
    eqrank = _lane_cumsum(eq.astype(jnp.int32), N) - eq.astype(jnp.int32)
    r = KI - jnp.sum(gt.astype(jnp.int32), axis=1, keepdims=True)
    sel = gt | (eq & (eqrank < r))
    seli = sel.astype(jnp.int32)
    rank = _lane_cumsum(seli, N) - seli
    jcol = lax.broadcasted_iota(jnp.int32, (KI, N), 0)
    a = (jnp.broadcast_to(rank, (KI, N)) == jcol) & jnp.broadcast_to(sel, (KI, N))
    icol = lax.broadcasted_iota(jnp.int32, (N, 1), 0).astype(jnp.float32)
    tkc = lax.dot_general(a.astype(jnp.float32), icol, (((1,), (0,)), ((), ())),
                          preferred_element_type=jnp.float32,
                          precision=lax.Precision.HIGHEST)
    tkc_ref[0] = tkc.astype(jnp.int32)


def _rel_topk_body(qtk_ref, ktk_ref, tk_ref, tkc_ref, st_ref, c_ref, o_ref):
    qt = qtk_ref[0]
    kt = ktk_ref[0]
    s = lax.dot_general(qt, kt, (((1,), (1,)), ((), ())),
                        preferred_element_type=jnp.float32)
    rows = lax.broadcasted_iota(jnp.int32, (KI, KI), 0)
    cols = lax.broadcasted_iota(jnp.int32, (KI, KI), 1)
    colsn = lax.broadcasted_iota(jnp.int32, (KI, N), 1)
    oh = (jnp.broadcast_to(tkc_ref[0], (KI, N)) == colsn).astype(jnp.float32)
    mz = lax.dot_general(oh, st_ref[0], (((1,), (0,)), ((), ())),
                         preferred_element_type=jnp.float32,
                         precision=lax.Precision.HIGHEST)
    p = jnp.exp(s - mz[:, 0:1]) / mz[:, 1:2]
    p = jnp.where(rows == cols, 1.0e9, p)
    tkb = jnp.broadcast_to(tk_ref[0], (KI, KI))

    slots = lax.broadcasted_iota(jnp.int32, (KI, KR), 1)

    def phase_a(i, carry):
        v, cmat, omat = carry
        m = jnp.max(v, axis=1, keepdims=True)
        first = jnp.min(jnp.where(v == m, cols, KI), axis=1, keepdims=True)
        hit = cols == first
        obj_i = jnp.min(jnp.where(hit, tkb, BIGI), axis=1, keepdims=True)
        cmat = jnp.where(slots == i, first, cmat)
        omat = jnp.where(slots == i, obj_i, omat)
        return jnp.where(hit, NEG, v), cmat, omat

    zm = jnp.zeros((KI, KR), jnp.int32)
    _, cmat, omat = lax.fori_loop(0, KR, phase_a, (p, zm, zm))

    def oe_round(r, mats):
        parity = r % 2
        out = []
        for mat in mats:
            right = jnp.roll(mat, -1, axis=1)
            left = jnp.roll(mat, 1, axis=1)
            lo = (slots % 2 == parity) & (slots < KR - 1)
            hi = (slots % 2 != parity) & (slots > 0)
            out.append(jnp.where(lo, jnp.minimum(mat, right),
                                 jnp.where(hi, jnp.maximum(mat, left), mat)))
        return tuple(out)

    cmat, omat = lax.fori_loop(0, KR, oe_round, (cmat, omat))
    c_ref[0] = cmat
    o_ref[0] = omat


def _relnorm_body(qtk_ref, c_ref, out_ref):
    rb = pl.program_id(1)
    qt = qtk_ref[0]
    cb = c_ref[0]
    t = lax.broadcasted_iota(jnp.int32, (RROWS, KR, KI), 2)
    onehot = (cb[:, :, None] == t).astype(jnp.float32).reshape(RROWS * KR, KI)
    obj = lax.dot_general(onehot, qt, (((1,), (0,)), ((), ())),
                          preferred_element_type=jnp.float32)
    qs = qtk_ref[0, pl.ds(rb * RROWS, RROWS)]
    subj = jnp.broadcast_to(qs[:, None, :],
                            (RROWS, KR, D)).reshape(RROWS * KR, D)
    rel = subj + obj
    mean = jnp.mean(rel, axis=1, keepdims=True)
    cen = rel - mean
    var = jnp.mean(cen * cen, axis=1, keepdims=True)
    out_ref[0] = cen / jnp.sqrt(var + 1e-5)


def _sc_gather(qf, kf, absidx):
    info = plsc.get_sparse_core_info()
    nc = info.num_cores
    nw = nc * info.num_subcores
    bpw = (B * KI) // nw
    mesh = plsc.VectorSubcoreMesh(core_axis_name="c", subcore_axis_name="s")

    @functools.partial(
        pl.kernel, mesh=mesh,
        out_type=[jax.ShapeDtypeStruct((B * KI, D), jnp.float32),
                  jax.ShapeDtypeStruct((B * KI, D), jnp.float32)],
        scratch_types=[pltpu.VMEM((bpw,), jnp.int32),
                       pltpu.VMEM((bpw, D), jnp.float32),
                       pltpu.SemaphoreType.DMA],
    )
    def gath(q_hbm, k_hbm, idx_hbm, qo_hbm, ko_hbm, idx_v, rows_v, sem):
        wid = lax.axis_index("s") * nc + lax.axis_index("c")
        base = wid * bpw
        pltpu.sync_copy(idx_hbm.at[pl.ds(base, bpw)], idx_v)
        pltpu.async_copy(q_hbm.at[idx_v], rows_v, sem).wait()
        pltpu.sync_copy(rows_v, qo_hbm.at[pl.ds(base, bpw)])
        pltpu.async_copy(k_hbm.at[idx_v], rows_v, sem).wait()
        pltpu.sync_copy(rows_v, ko_hbm.at[pl.ds(base, bpw)])

    return gath(qf, kf, absidx)


def kernel(q, k):
    scores, st = pl.pallas_call(
        _scores_diag_body,
        grid=(B, NRB),
        in_specs=[pl.BlockSpec((1, RB, D), lambda b, r: (b, r, 0)),
                  pl.BlockSpec((1, N, D), lambda b, r: (b, 0, 0))],
        out_specs=[pl.BlockSpec((1, RB, N), lambda b, r: (b, r, 0)),
                   pl.BlockSpec((1, 1, RB, 2), lambda b, r: (b, r, 0, 0))],
        out_shape=[jax.ShapeDtypeStruct((B, N, N), jnp.float32),
                   jax.ShapeDtypeStruct((B, NRB, RB, 2), jnp.float32)],
    )(q, k)
    st = st.reshape(B, N, 2)
    m = st[..., 0]
    d = st[..., 1]
    z = jnp.sum(jnp.exp(scores - m[:, :, None]), axis=-1)

    tkc = pl.pallas_call(
        _topk_diag_body,
        grid=(B,),
        in_specs=[pl.BlockSpec((1, 1, N), lambda b: (b, 0, 0)),
                  pl.BlockSpec((1, 1, N), lambda b: (b, 0, 0)),
                  pl.BlockSpec((1, 1, N), lambda b: (b, 0, 0))],
        out_specs=pl.BlockSpec((1, KI, 1), lambda b: (b, 0, 0)),
        out_shape=jax.ShapeDtypeStruct((B, KI, 1), jnp.int32),
    )(m.reshape(B, 1, N), d.reshape(B, 1, N), z.reshape(B, 1, N))
    tk = tkc.reshape(B, KI)

    absidx = (tk + N * jnp.arange(B, dtype=jnp.int32)[:, None]).reshape(B * KI)
    qtk_f, ktk_f = _sc_gather(q.reshape(B * N, D), k.reshape(B * N, D), absidx)
    qtk = qtk_f.reshape(B, KI, D)
    ktk = ktk_f.reshape(B, KI, D)

    c_ids, obj = pl.pallas_call(
        _rel_topk_body,
        grid=(B,),
        in_specs=[pl.BlockSpec((1, KI, D), lambda b: (b, 0, 0)),
                  pl.BlockSpec((1, KI, D), lambda b: (b, 0, 0)),
                  pl.BlockSpec((1, 1, KI), lambda b: (b, 0, 0)),
                  pl.BlockSpec((1, KI, 1), lambda b: (b, 0, 0)),
                  pl.BlockSpec((1, N, 2), lambda b: (b, 0, 0))],
        out_specs=[pl.BlockSpec((1, KI, KR), lambda b: (b, 0, 0)),
                   pl.BlockSpec((1, KI, KR), lambda b: (b, 0, 0))],
        out_shape=[jax.ShapeDtypeStruct((B, KI, KR), jnp.int32),
                   jax.ShapeDtypeStruct((B, KI, KR), jnp.int32)],
    )(qtk, ktk, tk.reshape(B, 1, KI), tkc, jnp.stack([m, z], axis=-1))

    rel_norm = pl.pallas_call(
        _relnorm_body,
        grid=(B, KI // RROWS),
        in_specs=[pl.BlockSpec((1, KI, D), lambda b, r: (b, 0, 0)),
                  pl.BlockSpec((1, RROWS, KR), lambda b, r: (b, r, 0))],
        out_specs=pl.BlockSpec((1, RROWS * KR, D), lambda b, r: (b, r, 0)),
        out_shape=jax.ShapeDtypeStruct((B, KI * KR, D), jnp.float32),
    )(qtk, c_ids)

    subj = jnp.broadcast_to(tk[:, :, None], (B, KI, KR)).reshape(B, KI * KR)
    objf = obj.reshape(B, KI * KR)
    bids = jnp.broadcast_to(jnp.arange(B, dtype=jnp.int32)[:, None],
                            (B, KI * KR))
    soi = jnp.stack([bids, subj, objf], axis=-1)
    return scores, soi, rel_norm
